# baseline (speedup 1.0000x reference)
import jax
import jax.numpy as jnp
from jax import lax
from jax.experimental import pallas as pl
from jax.experimental.pallas import tpu as pltpu
from jax.experimental.pallas import tpu_sc as plsc

N_NODES_K = 50000
N_EDGES_K = 800000
DH = 32
NC = 2
NS = 16
L = 16
CH = 128

N_PAD = 50048
DUMMY = N_PAD - 1
N_CHUNKS = N_PAD // CH
K = 2
EPT = 50176
E_PAD = NS * EPT
NCH_E = EPT // CH
NSUP = NCH_E // K


def _rsqrt16(v):
    nz = v > 0.0
    x = jnp.maximum(v, 1.0)
    s = 0.5 * (x + 1.0)
    for _ in range(15):
        s = 0.5 * (s + x / s)
    return jnp.where(nz, 1.0 / s, 0.0)


def _fill(ref, val):
    v = jnp.full((L,), val, dtype=ref.dtype)
    for g in range(CH // L):
        ref[pl.ds(g * L, L)] = v


def _body(table_hbm, src_p, dst_p, node_p, out_h,
          x0_h, x1_h, y_h, dis_h,
          idx2, idx4, msg4, wslab, dbuf, onesv, zvec,
          semi, semg, sems, acc_s):
    c = lax.axis_index("c")
    s = lax.axis_index("s")
    coff = c * N_PAD
    ebase = s * EPT

    _fill(onesv, 1.0)
    _fill(zvec, 0.0)

    zslab = msg4.at[0, 0]
    aslab = msg4.at[1, 0]
    yslab = msg4.at[1, 1]
    x0slab = msg4.at[0, 1]

    def fill_zslab():
        def zrow(r, _):
            zslab[r, pl.ds(0, L)] = jnp.zeros((L,), jnp.float32)
            zslab[r, pl.ds(L, L)] = jnp.zeros((L,), jnp.float32)
            return ()
        lax.fori_loop(0, CH, zrow, ())
    fill_zslab()

    def roundrobin(nch, fn, base=0):
        nk = (nch + NS - 1) // NS
        def body(k, _):
            ch = base + s + NS * k
            @pl.when(ch < base + nch)
            def _():
                fn(ch)
            return ()
        lax.fori_loop(0, nk, body, ())

    def load_dis(ch):
        pltpu.sync_copy(dis_h.at[pl.ds(coff + ch * CH, CH)], dbuf)

    def _bcast(ref, r):
        return plsc.load_gather(ref, [jnp.full((L,), r, jnp.int32)])

    import jax as _jax
    def zero_acc(ch):
        pltpu.sync_copy(zslab, acc_s.at[pl.ds(ch * CH, CH)])
    with _jax.named_scope("ph0_zero"):
        roundrobin(N_CHUNKS, zero_acc)

    with _jax.named_scope("ph1_deg"):
        oneslab = x0slab
        def orow(r, _):
            oneslab[r, pl.ds(0, L)] = jnp.zeros((L,), jnp.float32)
            oneslab[r, pl.ds(L, L)] = jnp.zeros((L,), jnp.float32)
            return ()
        lax.fori_loop(0, CH, orow, ())
        iota16 = lax.iota(jnp.int32, L)
        zeros16 = jnp.zeros((L,), jnp.int32)
        def ocol(g):
            plsc.store_scatter(oneslab, [g * L + iota16, zeros16],
                               jnp.full((L,), 1.0, jnp.float32))
        for g in range(CH // L):
            ocol(g)
        plsc.subcore_barrier()

        pltpu.async_copy(dst_p.at[pl.ds(ebase, CH)], idx4.at[0, 0, 1],
                         semi.at[0])

        def deg_step(j, _):
            p = lax.rem(j, 2)
            q = 1 - p
            pltpu.make_async_copy(dst_p.at[pl.ds(0, CH)], idx4.at[p, 0, 1],
                                  semi.at[p]).wait()
            @pl.when(j > 0)
            def _():
                pltpu.make_async_copy(oneslab, acc_s.at[idx4.at[q, 0, 1]],
                                      sems.at[q]).wait()
            @pl.when(j < NCH_E - 1)
            def _():
                pltpu.async_copy(dst_p.at[pl.ds(ebase + (j + 1) * CH, CH)],
                                 idx4.at[q, 0, 1], semi.at[q])
            pltpu.async_copy(oneslab, acc_s.at[idx4.at[p, 0, 1]],
                             sems.at[p], add=True)
            return ()
        lax.fori_loop(0, NCH_E, deg_step, ())
        pltpu.make_async_copy(oneslab, acc_s.at[idx4.at[(NCH_E - 1) % 2, 0, 1]],
                              sems.at[(NCH_E - 1) % 2]).wait()
        plsc.subcore_barrier()

        def dis_chunk(ch):
            pltpu.sync_copy(acc_s.at[pl.ds(ch * CH, CH)], aslab)
            for g in range(CH // L):
                col = plsc.load_gather(aslab, [g * L + iota16, zeros16])
                dbuf[pl.ds(g * L, L)] = _rsqrt16(col)
            pltpu.sync_copy(dbuf, dis_h.at[pl.ds(coff + ch * CH, CH)])
            pltpu.sync_copy(zslab, acc_s.at[pl.ds(ch * CH, CH)])
        roundrobin(N_CHUNKS, dis_chunk)
        plsc.subcore_barrier()

    def scale_rows(src_ref, dst_ref, extra=None):
        def row(r, _):
            d = _bcast(dbuf, r)
            for g in range(DH // L):
                sl = pl.ds(g * L, L)
                v = src_ref[r, sl] * d
                dst_ref[r, sl] = v
                if extra is not None:
                    extra[r, sl] = v * d
            return ()
        lax.fori_loop(0, CH, row, ())

    def extract_scale(off):
        def row(r, _):
            d = _bcast(dbuf, r)
            for g in range(DH // L):
                v = wslab[r, pl.ds(off + g * L, L)]
                x0slab[r, pl.ds(g * L, L)] = v
                yslab[r, pl.ds(g * L, L)] = v * d
            return ()
        lax.fori_loop(0, CH, row, ())

    def lookup_chunk(ch):
        pltpu.sync_copy(node_p.at[pl.ds(ch * CH, CH)], idx2.at[0])
        pltpu.sync_copy(table_hbm.at[idx2.at[0]], wslab)
        load_dis(ch)
        @pl.when(c == 0)
        def _():
            extract_scale(0)
        @pl.when(c == 1)
        def _():
            extract_scale(DH)
        pltpu.sync_copy(x0slab, x0_h.at[pl.ds(coff + ch * CH, CH)])
        pltpu.sync_copy(yslab, y_h.at[pl.ds(coff + ch * CH, CH)])
    with _jax.named_scope("ph2_lookup"):
        roundrobin(N_CHUNKS, lookup_chunk)
    plsc.subcore_barrier()

    def edge_phase():
        def forb(fn):
            def body(b, _):
                fn(b)
                return ()
            lax.fori_loop(0, K, body, ())

        def fire_idx(S, p):
            def f(b):
                base = ebase + (S * K + b) * CH
                pltpu.async_copy(src_p.at[pl.ds(base, CH)],
                                 idx4.at[p, b, 0], semi.at[p])
                pltpu.async_copy(dst_p.at[pl.ds(base, CH)],
                                 idx4.at[p, b, 1], semi.at[p])
            forb(f)

        def drain_idx(p):
            def f(b):
                pltpu.make_async_copy(src_p.at[pl.ds(0, CH)],
                                      idx4.at[p, b, 0], semi.at[p]).wait()
                pltpu.make_async_copy(dst_p.at[pl.ds(0, CH)],
                                      idx4.at[p, b, 1], semi.at[p]).wait()
            forb(f)

        def drain_scat(q):
            def f(b):
                pltpu.make_async_copy(msg4.at[q, b],
                                      acc_s.at[idx4.at[q, b, 1]],
                                      sems.at[q]).wait()
            forb(f)

        fire_idx(0, 0)

        def body(S, _):
            p = lax.rem(S, 2)
            q = 1 - p
            drain_idx(p)
            def off(b):
                for g in range(CH // L):
                    sl = pl.ds(g * L, L)
                    idx4[p, b, 0, sl] = idx4[p, b, 0, sl] + coff
            forb(off)
            forb(lambda b: pltpu.async_copy(y_h.at[idx4.at[p, b, 0]],
                                            msg4.at[p, b], semg.at[p]))
            @pl.when(S > 0)
            def _():
                drain_scat(q)
            @pl.when(S < NSUP - 1)
            def _():
                fire_idx(S + 1, q)
            forb(lambda b: pltpu.make_async_copy(
                y_h.at[idx4.at[p, b, 0]], msg4.at[p, b], semg.at[p]).wait())
            forb(lambda b: pltpu.async_copy(msg4.at[p, b],
                                            acc_s.at[idx4.at[p, b, 1]],
                                            sems.at[p], add=True))
            return ()
        lax.fori_loop(0, NSUP, body, ())
        drain_scat((NSUP - 1) % 2)

    with _jax.named_scope("ph3_edge1"):
        edge_phase()
    plsc.subcore_barrier()

    fill_zslab()
    def mid_chunk(ch):
        pltpu.sync_copy(acc_s.at[pl.ds(ch * CH, CH)], aslab)
        pltpu.sync_copy(zslab, acc_s.at[pl.ds(ch * CH, CH)])
        load_dis(ch)
        scale_rows(aslab, x0slab, extra=yslab)
        pltpu.sync_copy(x0slab, x1_h.at[pl.ds(coff + ch * CH, CH)])
        pltpu.sync_copy(yslab, y_h.at[pl.ds(coff + ch * CH, CH)])
    with _jax.named_scope("ph4_mid"):
        roundrobin(N_CHUNKS, mid_chunk)
    plsc.subcore_barrier()

    with _jax.named_scope("ph5_edge2"):
        edge_phase()
    plsc.subcore_barrier()

    def final_chunk(ch):
        pltpu.sync_copy(acc_s.at[pl.ds(ch * CH, CH)], aslab)
        pltpu.sync_copy(x0_h.at[pl.ds(coff + ch * CH, CH)], x0slab)
        pltpu.sync_copy(x1_h.at[pl.ds(coff + ch * CH, CH)], yslab)
        load_dis(ch)
        third = jnp.float32(1.0 / 3.0)
        def row(r, _):
            d = _bcast(dbuf, r)
            for g in range(DH // L):
                sl = pl.ds(g * L, L)
                v = (x0slab[r, sl] + yslab[r, sl] + aslab[r, sl] * d) * third
                zslab[r, sl] = v
            return ()
        lax.fori_loop(0, CH, row, ())
        pltpu.sync_copy(zslab, out_h.at[pl.ds(coff + ch * CH, CH)])
    with _jax.named_scope("ph6_final"):
        roundrobin(N_CHUNKS, final_chunk)


@jax.jit
def kernel(table, edge_index, node):
    src_p = jnp.pad(edge_index[0].astype(jnp.int32), (0, E_PAD - N_EDGES_K))
    dst_p = jnp.pad(edge_index[1].astype(jnp.int32), (0, E_PAD - N_EDGES_K),
                    constant_values=DUMMY)
    node_p = jnp.pad(node.astype(jnp.int32), (0, N_PAD - N_NODES_K))

    mesh = plsc.VectorSubcoreMesh(core_axis_name="c", subcore_axis_name="s")
    run = pl.kernel(
        _body,
        out_type=jax.ShapeDtypeStruct((NC * N_PAD, DH), jnp.float32),
        mesh=mesh,
        compiler_params=pltpu.CompilerParams(needs_layout_passes=False,
                                             use_tc_tiling_on_sc=False),
        scratch_types=[
            pltpu.HBM((NC * N_PAD, DH), jnp.float32),
            pltpu.HBM((NC * N_PAD, DH), jnp.float32),
            pltpu.HBM((NC * N_PAD, DH), jnp.float32),
            pltpu.HBM((NC * N_PAD,), jnp.float32),
            pltpu.VMEM((2, CH), jnp.int32),
            pltpu.VMEM((2, K, 2, CH), jnp.int32),
            pltpu.VMEM((2, K, CH, DH), jnp.float32),
            pltpu.VMEM((CH, 2 * DH), jnp.float32),
            pltpu.VMEM((CH,), jnp.float32),
            pltpu.VMEM((CH,), jnp.float32),
            pltpu.VMEM((CH,), jnp.float32),
            pltpu.SemaphoreType.DMA((2,)),
            pltpu.SemaphoreType.DMA((2,)),
            pltpu.SemaphoreType.DMA((2,)),
            pltpu.VMEM_SHARED((N_PAD, DH), jnp.float32),
        ],
    )
    o = run(table, src_p, dst_p, node_p)
    o = o.reshape(NC, N_PAD, DH)[:, :N_NODES_K]
    return jnp.concatenate([o[0], o[1]], axis=1)


# speedup vs baseline: 1.0394x; 1.0394x over previous
import jax
import jax.numpy as jnp
from jax import lax
from jax.experimental import pallas as pl
from jax.experimental.pallas import tpu as pltpu
from jax.experimental.pallas import tpu_sc as plsc

N_NODES_K = 50000
N_EDGES_K = 800000
DH = 32
NC = 2
NS = 16
L = 16
CH = 128

N_PAD = 50048
DUMMY = N_PAD - 1
N_CHUNKS = N_PAD // CH
K = 3
EPT = 50304
E_PAD = NS * EPT
NCH_E = EPT // CH
NSUP = NCH_E // K


def _rsqrt16(v):
    nz = v > 0.0
    x = jnp.maximum(v, 1.0)
    s = 0.5 * (x + 1.0)
    for _ in range(15):
        s = 0.5 * (s + x / s)
    return jnp.where(nz, 1.0 / s, 0.0)


def _fill(ref, val):
    v = jnp.full((L,), val, dtype=ref.dtype)
    for g in range(CH // L):
        ref[pl.ds(g * L, L)] = v


def _body(table_r, src_p, dst_p, node_p, out_h,
          x0_h, x1_h, y_h, dis_h,
          idx2, idx4, msg4, dbuf, onesv, zvec,
          semi, semg, sems, acc_s):
    c = lax.axis_index("c")
    s = lax.axis_index("s")
    coff = c * N_PAD
    ebase = s * EPT

    _fill(onesv, 1.0)
    _fill(zvec, 0.0)

    zslab = msg4.at[0, 0]
    aslab = msg4.at[1, 0]
    yslab = msg4.at[1, 1]
    x0slab = msg4.at[0, 1]
    xbuf = msg4.at[0, 2]

    def fill_zslab():
        def zrow(r, _):
            zslab[r, pl.ds(0, L)] = jnp.zeros((L,), jnp.float32)
            zslab[r, pl.ds(L, L)] = jnp.zeros((L,), jnp.float32)
            return ()
        lax.fori_loop(0, CH, zrow, ())
    fill_zslab()

    def roundrobin(nch, fn, base=0):
        nk = (nch + NS - 1) // NS
        def body(k, _):
            ch = base + s + NS * k
            @pl.when(ch < base + nch)
            def _():
                fn(ch)
            return ()
        lax.fori_loop(0, nk, body, ())

    def load_dis(ch):
        pltpu.sync_copy(dis_h.at[pl.ds(coff + ch * CH, CH)], dbuf)

    def _bcast(ref, r):
        return plsc.load_gather(ref, [jnp.full((L,), r, jnp.int32)])

    import jax as _jax
    def zero_acc(ch):
        pltpu.sync_copy(zslab, acc_s.at[pl.ds(ch * CH, CH)])
    with _jax.named_scope("ph0_zero"):
        roundrobin(N_CHUNKS, zero_acc)

    with _jax.named_scope("ph1_deg"):
        oneslab = x0slab
        def orow(r, _):
            oneslab[r, pl.ds(0, L)] = jnp.zeros((L,), jnp.float32)
            oneslab[r, pl.ds(L, L)] = jnp.zeros((L,), jnp.float32)
            return ()
        lax.fori_loop(0, CH, orow, ())
        iota16 = lax.iota(jnp.int32, L)
        zeros16 = jnp.zeros((L,), jnp.int32)
        def ocol(g):
            plsc.store_scatter(oneslab, [g * L + iota16, zeros16],
                               jnp.full((L,), 1.0, jnp.float32))
        for g in range(CH // L):
            ocol(g)
        plsc.subcore_barrier()

        pltpu.async_copy(dst_p.at[pl.ds(ebase, CH)], idx4.at[0, 0, 1],
                         semi.at[0])

        def deg_step(j, _):
            p = lax.rem(j, 2)
            q = 1 - p
            pltpu.make_async_copy(dst_p.at[pl.ds(0, CH)], idx4.at[p, 0, 1],
                                  semi.at[p]).wait()
            @pl.when(j > 0)
            def _():
                pltpu.make_async_copy(oneslab, acc_s.at[idx4.at[q, 0, 1]],
                                      sems.at[q]).wait()
            @pl.when(j < NCH_E - 1)
            def _():
                pltpu.async_copy(dst_p.at[pl.ds(ebase + (j + 1) * CH, CH)],
                                 idx4.at[q, 0, 1], semi.at[q])
            pltpu.async_copy(oneslab, acc_s.at[idx4.at[p, 0, 1]],
                             sems.at[p], add=True)
            return ()
        lax.fori_loop(0, NCH_E, deg_step, ())
        pltpu.make_async_copy(oneslab, acc_s.at[idx4.at[(NCH_E - 1) % 2, 0, 1]],
                              sems.at[(NCH_E - 1) % 2]).wait()
        plsc.subcore_barrier()

        def dis_chunk(ch):
            pltpu.sync_copy(acc_s.at[pl.ds(ch * CH, CH)], aslab)
            for g in range(CH // L):
                col = plsc.load_gather(aslab, [g * L + iota16, zeros16])
                dbuf[pl.ds(g * L, L)] = _rsqrt16(col)
            pltpu.sync_copy(dbuf, dis_h.at[pl.ds(coff + ch * CH, CH)])
            pltpu.sync_copy(zslab, acc_s.at[pl.ds(ch * CH, CH)])
        roundrobin(N_CHUNKS, dis_chunk)
        plsc.subcore_barrier()

    def scale_rows(src_ref, dst_ref, extra=None):
        def row(r, _):
            d = _bcast(dbuf, r)
            for g in range(DH // L):
                sl = pl.ds(g * L, L)
                v = src_ref[r, sl] * d
                dst_ref[r, sl] = v
                if extra is not None:
                    extra[r, sl] = v * d
            return ()
        lax.fori_loop(0, CH, row, ())

    def lookup_chunk(ch):
        pltpu.sync_copy(node_p.at[pl.ds(ch * CH, CH)], idx2.at[0])
        for g in range(CH // L):
            sl = pl.ds(g * L, L)
            idx2[0, sl] = idx2[0, sl] * 2 + c
        pltpu.sync_copy(table_r.at[idx2.at[0]], xbuf)
        load_dis(ch)
        scale_rows(xbuf, yslab)
        pltpu.sync_copy(xbuf, x0_h.at[pl.ds(coff + ch * CH, CH)])
        pltpu.sync_copy(yslab, y_h.at[pl.ds(coff + ch * CH, CH)])
    with _jax.named_scope("ph2_lookup"):
        roundrobin(N_CHUNKS, lookup_chunk)
    plsc.subcore_barrier()

    def edge_phase():
        def forb(fn):
            def body(b, _):
                fn(b)
                return ()
            lax.fori_loop(0, K, body, ())

        def fire_idx(S, p):
            def f(b):
                base = ebase + (S * K + b) * CH
                pltpu.async_copy(src_p.at[pl.ds(base, CH)],
                                 idx4.at[p, b, 0], semi.at[p])
                pltpu.async_copy(dst_p.at[pl.ds(base, CH)],
                                 idx4.at[p, b, 1], semi.at[p])
            forb(f)

        def drain_idx(p):
            def f(b):
                pltpu.make_async_copy(src_p.at[pl.ds(0, CH)],
                                      idx4.at[p, b, 0], semi.at[p]).wait()
                pltpu.make_async_copy(dst_p.at[pl.ds(0, CH)],
                                      idx4.at[p, b, 1], semi.at[p]).wait()
            forb(f)

        def drain_scat(q):
            def f(b):
                pltpu.make_async_copy(msg4.at[q, b],
                                      acc_s.at[idx4.at[q, b, 1]],
                                      sems.at[q]).wait()
            forb(f)

        fire_idx(0, 0)

        def body(S, _):
            p = lax.rem(S, 2)
            q = 1 - p
            drain_idx(p)
            def off(b):
                for g in range(CH // L):
                    sl = pl.ds(g * L, L)
                    idx4[p, b, 0, sl] = idx4[p, b, 0, sl] + coff
            forb(off)
            forb(lambda b: pltpu.async_copy(y_h.at[idx4.at[p, b, 0]],
                                            msg4.at[p, b], semg.at[p]))
            @pl.when(S > 0)
            def _():
                drain_scat(q)
            @pl.when(S < NSUP - 1)
            def _():
                fire_idx(S + 1, q)
            forb(lambda b: pltpu.make_async_copy(
                y_h.at[idx4.at[p, b, 0]], msg4.at[p, b], semg.at[p]).wait())
            forb(lambda b: pltpu.async_copy(msg4.at[p, b],
                                            acc_s.at[idx4.at[p, b, 1]],
                                            sems.at[p], add=True))
            return ()
        lax.fori_loop(0, NSUP, body, ())
        drain_scat((NSUP - 1) % 2)

    with _jax.named_scope("ph3_edge1"):
        edge_phase()
    plsc.subcore_barrier()

    fill_zslab()
    def mid_chunk(ch):
        pltpu.sync_copy(acc_s.at[pl.ds(ch * CH, CH)], aslab)
        pltpu.sync_copy(zslab, acc_s.at[pl.ds(ch * CH, CH)])
        load_dis(ch)
        scale_rows(aslab, x0slab, extra=yslab)
        pltpu.sync_copy(x0slab, x1_h.at[pl.ds(coff + ch * CH, CH)])
        pltpu.sync_copy(yslab, y_h.at[pl.ds(coff + ch * CH, CH)])
    with _jax.named_scope("ph4_mid"):
        roundrobin(N_CHUNKS, mid_chunk)
    plsc.subcore_barrier()

    with _jax.named_scope("ph5_edge2"):
        edge_phase()
    plsc.subcore_barrier()

    def final_chunk(ch):
        pltpu.sync_copy(acc_s.at[pl.ds(ch * CH, CH)], aslab)
        pltpu.sync_copy(x0_h.at[pl.ds(coff + ch * CH, CH)], x0slab)
        pltpu.sync_copy(x1_h.at[pl.ds(coff + ch * CH, CH)], yslab)
        load_dis(ch)
        third = jnp.float32(1.0 / 3.0)
        def row(r, _):
            d = _bcast(dbuf, r)
            for g in range(DH // L):
                sl = pl.ds(g * L, L)
                v = (x0slab[r, sl] + yslab[r, sl] + aslab[r, sl] * d) * third
                zslab[r, sl] = v
            return ()
        lax.fori_loop(0, CH, row, ())
        pltpu.sync_copy(zslab,
                        out_h.at[pl.ds(ch * CH, CH), pl.ds(c * DH, DH)])
    with _jax.named_scope("ph6_final"):
        roundrobin(N_CHUNKS, final_chunk)


@jax.jit
def kernel(table, edge_index, node):
    table_r = table.reshape(2 * 1000000, DH)
    src_p = jnp.pad(edge_index[0].astype(jnp.int32), (0, E_PAD - N_EDGES_K))
    dst_p = jnp.pad(edge_index[1].astype(jnp.int32), (0, E_PAD - N_EDGES_K),
                    constant_values=DUMMY)
    node_p = jnp.pad(node.astype(jnp.int32), (0, N_PAD - N_NODES_K))

    mesh = plsc.VectorSubcoreMesh(core_axis_name="c", subcore_axis_name="s")
    run = pl.kernel(
        _body,
        out_type=jax.ShapeDtypeStruct((N_PAD, NC * DH), jnp.float32),
        mesh=mesh,
        compiler_params=pltpu.CompilerParams(needs_layout_passes=False,
                                             use_tc_tiling_on_sc=False),
        scratch_types=[
            pltpu.HBM((NC * N_PAD, DH), jnp.float32),
            pltpu.HBM((NC * N_PAD, DH), jnp.float32),
            pltpu.HBM((NC * N_PAD, DH), jnp.float32),
            pltpu.HBM((NC * N_PAD,), jnp.float32),
            pltpu.VMEM((2, CH), jnp.int32),
            pltpu.VMEM((2, K, 2, CH), jnp.int32),
            pltpu.VMEM((2, K, CH, DH), jnp.float32),
            pltpu.VMEM((CH,), jnp.float32),
            pltpu.VMEM((CH,), jnp.float32),
            pltpu.VMEM((CH,), jnp.float32),
            pltpu.SemaphoreType.DMA((2,)),
            pltpu.SemaphoreType.DMA((2,)),
            pltpu.SemaphoreType.DMA((2,)),
            pltpu.VMEM_SHARED((N_PAD, DH), jnp.float32),
        ],
    )
    o = run(table_r, src_p, dst_p, node_p)
    return o[:N_NODES_K]


# speedup vs baseline: 1.1220x; 1.0794x over previous
import jax
import jax.numpy as jnp
from jax import lax
from jax.experimental import pallas as pl
from jax.experimental.pallas import tpu as pltpu
from jax.experimental.pallas import tpu_sc as plsc

N_NODES_K = 50000
N_EDGES_K = 800000
DH = 32
NC = 2
NS = 16
L = 16
CH = 128

N_PAD = 50048
DUMMY = N_PAD - 1
N_CHUNKS = N_PAD // CH
K = 3
EPT = N_EDGES_K // NS
NCH_E = 390
NSUP = NCH_E // K
TAIL = EPT - NCH_E * CH


def _rsqrt16(v):
    nz = v > 0.0
    x = jnp.maximum(v, 1.0)
    s = 0.5 * (x + 1.0)
    for _ in range(15):
        s = 0.5 * (s + x / s)
    return jnp.where(nz, 1.0 / s, 0.0)


def _fill(ref, val):
    v = jnp.full((L,), val, dtype=ref.dtype)
    for g in range(CH // L):
        ref[pl.ds(g * L, L)] = v


def _body(table_r, src_p, dst_p, node_p, out_h,
          x0_h, x1_h, y_h, dis_h,
          idx2, idx4, msg4, dbuf, onesv, zvec,
          semi, semg, sems, acc_s):
    c = lax.axis_index("c")
    s = lax.axis_index("s")
    coff = c * N_PAD
    ebase = s * EPT

    _fill(onesv, 1.0)
    _fill(zvec, 0.0)

    zslab = msg4.at[0, 0]
    aslab = msg4.at[1, 0]
    yslab = msg4.at[1, 1]
    x0slab = msg4.at[0, 1]
    xbuf = msg4.at[0, 2]

    def fill_zslab():
        def zrow(r, _):
            zslab[r, pl.ds(0, L)] = jnp.zeros((L,), jnp.float32)
            zslab[r, pl.ds(L, L)] = jnp.zeros((L,), jnp.float32)
            return ()
        lax.fori_loop(0, CH, zrow, ())
    fill_zslab()

    def roundrobin(nch, fn, base=0):
        nk = (nch + NS - 1) // NS
        def body(k, _):
            ch = base + s + NS * k
            @pl.when(ch < base + nch)
            def _():
                fn(ch)
            return ()
        lax.fori_loop(0, nk, body, ())

    def load_dis(ch):
        pltpu.sync_copy(dis_h.at[pl.ds(coff + ch * CH, CH)], dbuf)

    def _bcast(ref, r):
        return plsc.load_gather(ref, [jnp.full((L,), r, jnp.int32)])

    def load_tail_idx(ref_hbm, row, padval):
        pltpu.sync_copy(ref_hbm.at[pl.ds(ebase + NCH_E * CH, TAIL)],
                        idx2.at[row, pl.ds(0, TAIL)])
        pv = jnp.full((L,), padval, jnp.int32)
        for o in range(TAIL, CH, L):
            idx2[row, pl.ds(o, L)] = pv

    import jax as _jax
    def zero_acc(ch):
        pltpu.sync_copy(zslab, acc_s.at[pl.ds(ch * CH, CH)])
    with _jax.named_scope("ph0_zero"):
        roundrobin(N_CHUNKS, zero_acc)

    with _jax.named_scope("ph1_deg"):
        oneslab = x0slab
        def orow(r, _):
            oneslab[r, pl.ds(0, L)] = jnp.zeros((L,), jnp.float32)
            oneslab[r, pl.ds(L, L)] = jnp.zeros((L,), jnp.float32)
            return ()
        lax.fori_loop(0, CH, orow, ())
        iota16 = lax.iota(jnp.int32, L)
        zeros16 = jnp.zeros((L,), jnp.int32)
        def ocol(g):
            plsc.store_scatter(oneslab, [g * L + iota16, zeros16],
                               jnp.full((L,), 1.0, jnp.float32))
        for g in range(CH // L):
            ocol(g)
        plsc.subcore_barrier()

        pltpu.async_copy(dst_p.at[pl.ds(ebase, CH)], idx4.at[0, 0, 1],
                         semi.at[0])

        def deg_step(j, _):
            p = lax.rem(j, 2)
            q = 1 - p
            pltpu.make_async_copy(dst_p.at[pl.ds(0, CH)], idx4.at[p, 0, 1],
                                  semi.at[p]).wait()
            @pl.when(j > 0)
            def _():
                pltpu.make_async_copy(oneslab, acc_s.at[idx4.at[q, 0, 1]],
                                      sems.at[q]).wait()
            @pl.when(j < NCH_E - 1)
            def _():
                pltpu.async_copy(dst_p.at[pl.ds(ebase + (j + 1) * CH, CH)],
                                 idx4.at[q, 0, 1], semi.at[q])
            pltpu.async_copy(oneslab, acc_s.at[idx4.at[p, 0, 1]],
                             sems.at[p], add=True)
            return ()
        lax.fori_loop(0, NCH_E, deg_step, ())
        pltpu.make_async_copy(oneslab, acc_s.at[idx4.at[(NCH_E - 1) % 2, 0, 1]],
                              sems.at[(NCH_E - 1) % 2]).wait()
        load_tail_idx(dst_p, 1, DUMMY)
        pltpu.sync_copy(oneslab, acc_s.at[idx2.at[1]], add=True)
        plsc.subcore_barrier()

        def dis_chunk(ch):
            pltpu.sync_copy(acc_s.at[pl.ds(ch * CH, CH)], aslab)
            for g in range(CH // L):
                col = plsc.load_gather(aslab, [g * L + iota16, zeros16])
                dbuf[pl.ds(g * L, L)] = _rsqrt16(col)
            pltpu.sync_copy(dbuf, dis_h.at[pl.ds(coff + ch * CH, CH)])
            pltpu.sync_copy(zslab, acc_s.at[pl.ds(ch * CH, CH)])
        roundrobin(N_CHUNKS, dis_chunk)
        plsc.subcore_barrier()

    def scale_rows(src_ref, dst_ref, extra=None):
        def row(r, _):
            d = _bcast(dbuf, r)
            for g in range(DH // L):
                sl = pl.ds(g * L, L)
                v = src_ref[r, sl] * d
                dst_ref[r, sl] = v
                if extra is not None:
                    extra[r, sl] = v * d
            return ()
        lax.fori_loop(0, CH, row, ())

    def lookup_chunk(ch):
        @pl.when(ch < N_CHUNKS - 1)
        def _():
            pltpu.sync_copy(node_p.at[pl.ds(ch * CH, CH)], idx2.at[0])
        @pl.when(ch == N_CHUNKS - 1)
        def _():
            pltpu.sync_copy(node_p.at[pl.ds(ch * CH, TAIL)],
                            idx2.at[0, pl.ds(0, TAIL)])
            zv = jnp.zeros((L,), jnp.int32)
            for o in range(TAIL, CH, L):
                idx2[0, pl.ds(o, L)] = zv
        for g in range(CH // L):
            sl = pl.ds(g * L, L)
            idx2[0, sl] = idx2[0, sl] * 2 + c
        pltpu.sync_copy(table_r.at[idx2.at[0]], xbuf)
        load_dis(ch)
        scale_rows(xbuf, yslab)
        pltpu.sync_copy(xbuf, x0_h.at[pl.ds(coff + ch * CH, CH)])
        pltpu.sync_copy(yslab, y_h.at[pl.ds(coff + ch * CH, CH)])
    with _jax.named_scope("ph2_lookup"):
        roundrobin(N_CHUNKS, lookup_chunk)
    plsc.subcore_barrier()

    def edge_phase():
        def forb(fn):
            def body(b, _):
                fn(b)
                return ()
            lax.fori_loop(0, K, body, ())

        def fire_idx(S, p):
            def f(b):
                base = ebase + (S * K + b) * CH
                pltpu.async_copy(src_p.at[pl.ds(base, CH)],
                                 idx4.at[p, b, 0], semi.at[p])
                pltpu.async_copy(dst_p.at[pl.ds(base, CH)],
                                 idx4.at[p, b, 1], semi.at[p])
            forb(f)

        def drain_idx(p):
            def f(b):
                pltpu.make_async_copy(src_p.at[pl.ds(0, CH)],
                                      idx4.at[p, b, 0], semi.at[p]).wait()
                pltpu.make_async_copy(dst_p.at[pl.ds(0, CH)],
                                      idx4.at[p, b, 1], semi.at[p]).wait()
            forb(f)

        def drain_scat(q):
            def f(b):
                pltpu.make_async_copy(msg4.at[q, b],
                                      acc_s.at[idx4.at[q, b, 1]],
                                      sems.at[q]).wait()
            forb(f)

        fire_idx(0, 0)

        def body(S, _):
            p = lax.rem(S, 2)
            q = 1 - p
            drain_idx(p)
            def off(b):
                for g in range(CH // L):
                    sl = pl.ds(g * L, L)
                    idx4[p, b, 0, sl] = idx4[p, b, 0, sl] + coff
            forb(off)
            forb(lambda b: pltpu.async_copy(y_h.at[idx4.at[p, b, 0]],
                                            msg4.at[p, b], semg.at[p]))
            @pl.when(S > 0)
            def _():
                drain_scat(q)
            @pl.when(S < NSUP - 1)
            def _():
                fire_idx(S + 1, q)
            forb(lambda b: pltpu.make_async_copy(
                y_h.at[idx4.at[p, b, 0]], msg4.at[p, b], semg.at[p]).wait())
            forb(lambda b: pltpu.async_copy(msg4.at[p, b],
                                            acc_s.at[idx4.at[p, b, 1]],
                                            sems.at[p], add=True))
            return ()
        lax.fori_loop(0, NSUP, body, ())
        drain_scat((NSUP - 1) % 2)
        load_tail_idx(src_p, 0, 0)
        load_tail_idx(dst_p, 1, DUMMY)
        for g in range(CH // L):
            sl = pl.ds(g * L, L)
            idx2[0, sl] = idx2[0, sl] + coff
        pltpu.sync_copy(y_h.at[idx2.at[0]], aslab)
        pltpu.sync_copy(aslab, acc_s.at[idx2.at[1]], add=True)

    with _jax.named_scope("ph3_edge1"):
        edge_phase()
    plsc.subcore_barrier()

    fill_zslab()
    def mid_chunk(ch):
        pltpu.sync_copy(acc_s.at[pl.ds(ch * CH, CH)], aslab)
        pltpu.sync_copy(zslab, acc_s.at[pl.ds(ch * CH, CH)])
        load_dis(ch)
        scale_rows(aslab, x0slab, extra=yslab)
        pltpu.sync_copy(x0slab, x1_h.at[pl.ds(coff + ch * CH, CH)])
        pltpu.sync_copy(yslab, y_h.at[pl.ds(coff + ch * CH, CH)])
    with _jax.named_scope("ph4_mid"):
        roundrobin(N_CHUNKS, mid_chunk)
    plsc.subcore_barrier()

    with _jax.named_scope("ph5_edge2"):
        edge_phase()
    plsc.subcore_barrier()

    def final_chunk(ch):
        pltpu.sync_copy(acc_s.at[pl.ds(ch * CH, CH)], aslab)
        pltpu.sync_copy(x0_h.at[pl.ds(coff + ch * CH, CH)], x0slab)
        pltpu.sync_copy(x1_h.at[pl.ds(coff + ch * CH, CH)], yslab)
        load_dis(ch)
        third = jnp.float32(1.0 / 3.0)
        def row(r, _):
            d = _bcast(dbuf, r)
            for g in range(DH // L):
                sl = pl.ds(g * L, L)
                v = (x0slab[r, sl] + yslab[r, sl] + aslab[r, sl] * d) * third
                zslab[r, sl] = v
            return ()
        lax.fori_loop(0, CH, row, ())
        pltpu.sync_copy(zslab,
                        out_h.at[pl.ds(ch * CH, CH), pl.ds(c * DH, DH)])
    with _jax.named_scope("ph6_final"):
        roundrobin(N_CHUNKS, final_chunk)


@jax.jit
def kernel(table, edge_index, node):
    table_r = table.reshape(2 * 1000000, DH)
    src_p = edge_index[0]
    dst_p = edge_index[1]

    mesh = plsc.VectorSubcoreMesh(core_axis_name="c", subcore_axis_name="s")
    run = pl.kernel(
        _body,
        out_type=jax.ShapeDtypeStruct((N_PAD, NC * DH), jnp.float32),
        mesh=mesh,
        compiler_params=pltpu.CompilerParams(needs_layout_passes=False,
                                             use_tc_tiling_on_sc=False),
        scratch_types=[
            pltpu.HBM((NC * N_PAD, DH), jnp.float32),
            pltpu.HBM((NC * N_PAD, DH), jnp.float32),
            pltpu.HBM((NC * N_PAD, DH), jnp.float32),
            pltpu.HBM((NC * N_PAD,), jnp.float32),
            pltpu.VMEM((2, CH), jnp.int32),
            pltpu.VMEM((2, K, 2, CH), jnp.int32),
            pltpu.VMEM((2, K, CH, DH), jnp.float32),
            pltpu.VMEM((CH,), jnp.float32),
            pltpu.VMEM((CH,), jnp.float32),
            pltpu.VMEM((CH,), jnp.float32),
            pltpu.SemaphoreType.DMA((2,)),
            pltpu.SemaphoreType.DMA((2,)),
            pltpu.SemaphoreType.DMA((2,)),
            pltpu.VMEM_SHARED((N_PAD, DH), jnp.float32),
        ],
    )
    o = run(table_r, src_p, dst_p, node)
    return o[:N_NODES_K]


# speedup vs baseline: 1.1315x; 1.0085x over previous
import jax
import jax.numpy as jnp
from jax import lax
from jax.experimental import pallas as pl
from jax.experimental.pallas import tpu as pltpu
from jax.experimental.pallas import tpu_sc as plsc

N_NODES_K = 50000
N_EDGES_K = 800000
DH = 32
NC = 2
NS = 16
L = 16
CH = 128

N_PAD = 50048
DUMMY = N_PAD - 1
N_CHUNKS = N_PAD // CH
K = 3
EPT = N_EDGES_K // NS
NCH_E = 390
NSUP = NCH_E // K
TAIL = EPT - NCH_E * CH


def _rsqrt16(v):
    nz = v > 0.0
    x = jnp.maximum(v, 1.0)
    s = 0.5 * (x + 1.0)
    for _ in range(15):
        s = 0.5 * (s + x / s)
    return jnp.where(nz, 1.0 / s, 0.0)


def _fill(ref, val):
    v = jnp.full((L,), val, dtype=ref.dtype)
    for g in range(CH // L):
        ref[pl.ds(g * L, L)] = v


def _body(table_r, src_p, dst_p, node_p, out_h,
          x0_h, x1_h, y_h, dis_h,
          idx2, idx4, msg4, dbuf, onesv, zvec,
          semi, semg, sems, acc_s):
    c = lax.axis_index("c")
    s = lax.axis_index("s")
    coff = c * N_PAD
    ebase = s * EPT

    _fill(onesv, 1.0)
    _fill(zvec, 0.0)

    zslab = msg4.at[0, 0]
    aslab = msg4.at[1, 0]
    yslab = msg4.at[1, 1]
    x0slab = msg4.at[0, 1]
    xbuf = msg4.at[0, 2]

    def fill_zslab():
        def zrow(r, _):
            zslab[r, pl.ds(0, L)] = jnp.zeros((L,), jnp.float32)
            zslab[r, pl.ds(L, L)] = jnp.zeros((L,), jnp.float32)
            return ()
        lax.fori_loop(0, CH, zrow, ())
    fill_zslab()

    def roundrobin(nch, fn, base=0):
        nk = (nch + NS - 1) // NS
        def body(k, _):
            ch = base + s + NS * k
            @pl.when(ch < base + nch)
            def _():
                fn(ch)
            return ()
        lax.fori_loop(0, nk, body, ())

    def load_dis(ch):
        pltpu.sync_copy(dis_h.at[pl.ds(coff + ch * CH, CH)], dbuf)

    def _bcast(ref, r):
        return plsc.load_gather(ref, [jnp.full((L,), r, jnp.int32)])

    def load_tail_idx(ref_hbm, row, padval):
        pltpu.sync_copy(ref_hbm.at[pl.ds(ebase + NCH_E * CH, TAIL)],
                        idx2.at[row, pl.ds(0, TAIL)])
        pv = jnp.full((L,), padval, jnp.int32)
        for o in range(TAIL, CH, L):
            idx2[row, pl.ds(o, L)] = pv

    import jax as _jax
    def zero_acc(ch):
        pltpu.sync_copy(zslab, acc_s.at[pl.ds(ch * CH, CH)])
    with _jax.named_scope("ph0_zero"):
        roundrobin(N_CHUNKS, zero_acc)

    def scale_rows(src_ref, dst_ref, extra=None):
        def row(r, _):
            d = _bcast(dbuf, r)
            for g in range(DH // L):
                sl = pl.ds(g * L, L)
                v = src_ref[r, sl] * d
                dst_ref[r, sl] = v
                if extra is not None:
                    extra[r, sl] = v * d
            return ()
        lax.fori_loop(0, CH, row, ())

    def lookup_chunk(ch):
        @pl.when(ch < N_CHUNKS - 1)
        def _():
            pltpu.sync_copy(node_p.at[pl.ds(ch * CH, CH)], idx2.at[0])
        @pl.when(ch == N_CHUNKS - 1)
        def _():
            pltpu.sync_copy(node_p.at[pl.ds(ch * CH, TAIL)],
                            idx2.at[0, pl.ds(0, TAIL)])
            zv = jnp.zeros((L,), jnp.int32)
            for o in range(TAIL, CH, L):
                idx2[0, pl.ds(o, L)] = zv
        for g in range(CH // L):
            sl = pl.ds(g * L, L)
            idx2[0, sl] = idx2[0, sl] * 2 + c
        pltpu.sync_copy(table_r.at[idx2.at[0]], xbuf)
        scale_rows(xbuf, yslab)
        pltpu.sync_copy(xbuf, x0_h.at[pl.ds(coff + ch * CH, CH)])
        pltpu.sync_copy(yslab, y_h.at[pl.ds(coff + ch * CH, CH)])

    with _jax.named_scope("ph1_deg"):
        oneslab = x0slab
        def orow(r, _):
            oneslab[r, pl.ds(0, L)] = jnp.zeros((L,), jnp.float32)
            oneslab[r, pl.ds(L, L)] = jnp.zeros((L,), jnp.float32)
            return ()
        lax.fori_loop(0, CH, orow, ())
        iota16 = lax.iota(jnp.int32, L)
        zeros16 = jnp.zeros((L,), jnp.int32)
        def ocol(g):
            plsc.store_scatter(oneslab, [g * L + iota16, zeros16],
                               jnp.full((L,), 1.0, jnp.float32))
        for g in range(CH // L):
            ocol(g)
        plsc.subcore_barrier()

        pltpu.async_copy(dst_p.at[pl.ds(ebase, CH)], idx4.at[0, 0, 1],
                         semi.at[0])

        def deg_step(j, _):
            p = lax.rem(j, 2)
            q = 1 - p
            pltpu.make_async_copy(dst_p.at[pl.ds(0, CH)], idx4.at[p, 0, 1],
                                  semi.at[p]).wait()
            @pl.when(j > 0)
            def _():
                pltpu.make_async_copy(oneslab, acc_s.at[idx4.at[q, 0, 1]],
                                      sems.at[q]).wait()
            @pl.when(j < NCH_E - 1)
            def _():
                pltpu.async_copy(dst_p.at[pl.ds(ebase + (j + 1) * CH, CH)],
                                 idx4.at[q, 0, 1], semi.at[q])
            pltpu.async_copy(oneslab, acc_s.at[idx4.at[p, 0, 1]],
                             sems.at[p], add=True)
            return ()
        lax.fori_loop(0, NCH_E, deg_step, ())
        pltpu.make_async_copy(oneslab, acc_s.at[idx4.at[(NCH_E - 1) % 2, 0, 1]],
                              sems.at[(NCH_E - 1) % 2]).wait()
        load_tail_idx(dst_p, 1, DUMMY)
        pltpu.sync_copy(oneslab, acc_s.at[idx2.at[1]], add=True)
        plsc.subcore_barrier()

        def dis_chunk(ch):
            pltpu.sync_copy(acc_s.at[pl.ds(ch * CH, CH)], aslab)
            for g in range(CH // L):
                col = plsc.load_gather(aslab, [g * L + iota16, zeros16])
                dbuf[pl.ds(g * L, L)] = _rsqrt16(col)
            pltpu.sync_copy(dbuf, dis_h.at[pl.ds(coff + ch * CH, CH)])
            pltpu.sync_copy(zslab, acc_s.at[pl.ds(ch * CH, CH)])
            lookup_chunk(ch)
        roundrobin(N_CHUNKS, dis_chunk)
        plsc.subcore_barrier()

    def edge_phase():
        def forb(fn):
            def body(b, _):
                fn(b)
                return ()
            lax.fori_loop(0, K, body, ())

        def fire_idx(S, p):
            def f(b):
                base = ebase + (S * K + b) * CH
                pltpu.async_copy(src_p.at[pl.ds(base, CH)],
                                 idx4.at[p, b, 0], semi.at[p])
                pltpu.async_copy(dst_p.at[pl.ds(base, CH)],
                                 idx4.at[p, b, 1], semi.at[p])
            forb(f)

        def drain_idx(p):
            def f(b):
                pltpu.make_async_copy(src_p.at[pl.ds(0, CH)],
                                      idx4.at[p, b, 0], semi.at[p]).wait()
                pltpu.make_async_copy(dst_p.at[pl.ds(0, CH)],
                                      idx4.at[p, b, 1], semi.at[p]).wait()
            forb(f)

        def drain_scat(q):
            def f(b):
                pltpu.make_async_copy(msg4.at[q, b],
                                      acc_s.at[idx4.at[q, b, 1]],
                                      sems.at[q]).wait()
            forb(f)

        fire_idx(0, 0)

        def body(S, _):
            p = lax.rem(S, 2)
            q = 1 - p
            drain_idx(p)
            def off(b):
                for g in range(CH // L):
                    sl = pl.ds(g * L, L)
                    idx4[p, b, 0, sl] = idx4[p, b, 0, sl] + coff
            forb(off)
            forb(lambda b: pltpu.async_copy(y_h.at[idx4.at[p, b, 0]],
                                            msg4.at[p, b], semg.at[p]))
            @pl.when(S > 0)
            def _():
                drain_scat(q)
            @pl.when(S < NSUP - 1)
            def _():
                fire_idx(S + 1, q)
            forb(lambda b: pltpu.make_async_copy(
                y_h.at[idx4.at[p, b, 0]], msg4.at[p, b], semg.at[p]).wait())
            forb(lambda b: pltpu.async_copy(msg4.at[p, b],
                                            acc_s.at[idx4.at[p, b, 1]],
                                            sems.at[p], add=True))
            return ()
        lax.fori_loop(0, NSUP, body, ())
        drain_scat((NSUP - 1) % 2)
        load_tail_idx(src_p, 0, 0)
        load_tail_idx(dst_p, 1, DUMMY)
        for g in range(CH // L):
            sl = pl.ds(g * L, L)
            idx2[0, sl] = idx2[0, sl] + coff
        pltpu.sync_copy(y_h.at[idx2.at[0]], aslab)
        pltpu.sync_copy(aslab, acc_s.at[idx2.at[1]], add=True)

    with _jax.named_scope("ph3_edge1"):
        edge_phase()
    plsc.subcore_barrier()

    fill_zslab()
    def mid_chunk(ch):
        pltpu.sync_copy(acc_s.at[pl.ds(ch * CH, CH)], aslab)
        pltpu.sync_copy(zslab, acc_s.at[pl.ds(ch * CH, CH)])
        load_dis(ch)
        scale_rows(aslab, x0slab, extra=yslab)
        pltpu.sync_copy(x0slab, x1_h.at[pl.ds(coff + ch * CH, CH)])
        pltpu.sync_copy(yslab, y_h.at[pl.ds(coff + ch * CH, CH)])
    with _jax.named_scope("ph4_mid"):
        roundrobin(N_CHUNKS, mid_chunk)
    plsc.subcore_barrier()

    with _jax.named_scope("ph5_edge2"):
        edge_phase()
    plsc.subcore_barrier()

    def final_chunk(ch):
        pltpu.sync_copy(acc_s.at[pl.ds(ch * CH, CH)], aslab)
        pltpu.sync_copy(x0_h.at[pl.ds(coff + ch * CH, CH)], x0slab)
        pltpu.sync_copy(x1_h.at[pl.ds(coff + ch * CH, CH)], yslab)
        load_dis(ch)
        third = jnp.float32(1.0 / 3.0)
        def row(r, _):
            d = _bcast(dbuf, r)
            for g in range(DH // L):
                sl = pl.ds(g * L, L)
                v = (x0slab[r, sl] + yslab[r, sl] + aslab[r, sl] * d) * third
                zslab[r, sl] = v
            return ()
        lax.fori_loop(0, CH, row, ())
        pltpu.sync_copy(zslab,
                        out_h.at[pl.ds(ch * CH, CH), pl.ds(c * DH, DH)])
    with _jax.named_scope("ph6_final"):
        roundrobin(N_CHUNKS, final_chunk)


@jax.jit
def kernel(table, edge_index, node):
    table_r = table.reshape(2 * 1000000, DH)
    src_p = edge_index[0]
    dst_p = edge_index[1]

    mesh = plsc.VectorSubcoreMesh(core_axis_name="c", subcore_axis_name="s")
    run = pl.kernel(
        _body,
        out_type=jax.ShapeDtypeStruct((N_PAD, NC * DH), jnp.float32),
        mesh=mesh,
        compiler_params=pltpu.CompilerParams(needs_layout_passes=False,
                                             use_tc_tiling_on_sc=False),
        scratch_types=[
            pltpu.HBM((NC * N_PAD, DH), jnp.float32),
            pltpu.HBM((NC * N_PAD, DH), jnp.float32),
            pltpu.HBM((NC * N_PAD, DH), jnp.float32),
            pltpu.HBM((NC * N_PAD,), jnp.float32),
            pltpu.VMEM((2, CH), jnp.int32),
            pltpu.VMEM((2, K, 2, CH), jnp.int32),
            pltpu.VMEM((2, K, CH, DH), jnp.float32),
            pltpu.VMEM((CH,), jnp.float32),
            pltpu.VMEM((CH,), jnp.float32),
            pltpu.VMEM((CH,), jnp.float32),
            pltpu.SemaphoreType.DMA((2,)),
            pltpu.SemaphoreType.DMA((2,)),
            pltpu.SemaphoreType.DMA((2,)),
            pltpu.VMEM_SHARED((N_PAD, DH), jnp.float32),
        ],
    )
    o = run(table_r, src_p, dst_p, node)
    return o[:N_NODES_K]


# speedup vs baseline: 1.1321x; 1.0005x over previous
import jax
import jax.numpy as jnp
from jax import lax
from jax.experimental import pallas as pl
from jax.experimental.pallas import tpu as pltpu
from jax.experimental.pallas import tpu_sc as plsc

N_NODES_K = 50000
N_EDGES_K = 800000
DH = 32
NC = 2
NS = 16
L = 16
CH = 128

N_PAD = 50048
DUMMY = N_PAD - 1
N_CHUNKS = N_PAD // CH
K = 3
EPT = N_EDGES_K // NS
NCH_E = 390
NSUP = NCH_E // K
TAIL = EPT - NCH_E * CH


def _rsqrt16(v):
    nz = v > 0.0
    x = jnp.maximum(v, 1.0)
    s = 0.5 * (x + 1.0)
    for _ in range(15):
        s = 0.5 * (s + x / s)
    return jnp.where(nz, 1.0 / s, 0.0)


def _fill(ref, val):
    v = jnp.full((L,), val, dtype=ref.dtype)
    for g in range(CH // L):
        ref[pl.ds(g * L, L)] = v


def _body(table_r, src_p, dst_p, node_p, out_h,
          x0_h, x1_h, y_h, dis_h,
          idx2, idx4, msg4, dbuf, onesv, zvec,
          semi, semg, sems, acc_s):
    c = lax.axis_index("c")
    s = lax.axis_index("s")
    coff = c * N_PAD
    ebase = s * EPT

    _fill(onesv, 1.0)
    _fill(zvec, 0.0)

    zslab = msg4.at[0, 0]
    aslab = msg4.at[1, 0]
    yslab = msg4.at[1, 1]
    x0slab = msg4.at[0, 1]
    xbuf = msg4.at[0, 2]

    def fill_zslab():
        def zrow(r, _):
            zslab[r, pl.ds(0, L)] = jnp.zeros((L,), jnp.float32)
            zslab[r, pl.ds(L, L)] = jnp.zeros((L,), jnp.float32)
            return ()
        lax.fori_loop(0, CH, zrow, ())
    fill_zslab()

    def roundrobin(nch, fn, base=0):
        nk = (nch + NS - 1) // NS
        def body(k, _):
            ch = base + s + NS * k
            @pl.when(ch < base + nch)
            def _():
                fn(ch)
            return ()
        lax.fori_loop(0, nk, body, ())

    def load_dis(ch):
        pltpu.sync_copy(dis_h.at[pl.ds(coff + ch * CH, CH)], dbuf)

    def _bcast(ref, r):
        return plsc.load_gather(ref, [jnp.full((L,), r, jnp.int32)])

    def load_tail_idx(ref_hbm, row, padval):
        pltpu.sync_copy(ref_hbm.at[pl.ds(ebase + NCH_E * CH, TAIL)],
                        idx2.at[row, pl.ds(0, TAIL)])
        pv = jnp.full((L,), padval, jnp.int32)
        for o in range(TAIL, CH, L):
            idx2[row, pl.ds(o, L)] = pv

    def zero_acc(ch):
        pltpu.sync_copy(zslab, acc_s.at[pl.ds(ch * CH, CH)])
    roundrobin(N_CHUNKS, zero_acc)

    def scale_rows(src_ref, dst_ref, extra=None):
        def row(r, _):
            d = _bcast(dbuf, r)
            for g in range(DH // L):
                sl = pl.ds(g * L, L)
                v = src_ref[r, sl] * d
                dst_ref[r, sl] = v
                if extra is not None:
                    extra[r, sl] = v * d
            return ()
        lax.fori_loop(0, CH, row, ())

    def lookup_chunk(ch):
        @pl.when(ch < N_CHUNKS - 1)
        def _():
            pltpu.sync_copy(node_p.at[pl.ds(ch * CH, CH)], idx2.at[0])
        @pl.when(ch == N_CHUNKS - 1)
        def _():
            pltpu.sync_copy(node_p.at[pl.ds(ch * CH, TAIL)],
                            idx2.at[0, pl.ds(0, TAIL)])
            zv = jnp.zeros((L,), jnp.int32)
            for o in range(TAIL, CH, L):
                idx2[0, pl.ds(o, L)] = zv
        for g in range(CH // L):
            sl = pl.ds(g * L, L)
            idx2[0, sl] = idx2[0, sl] * 2 + c
        pltpu.sync_copy(table_r.at[idx2.at[0]], xbuf)
        scale_rows(xbuf, yslab)
        pltpu.sync_copy(xbuf, x0_h.at[pl.ds(coff + ch * CH, CH)])
        pltpu.sync_copy(yslab, y_h.at[pl.ds(coff + ch * CH, CH)])

    oneslab = x0slab
    def orow(r, _):
        oneslab[r, pl.ds(0, L)] = jnp.zeros((L,), jnp.float32)
        oneslab[r, pl.ds(L, L)] = jnp.zeros((L,), jnp.float32)
        return ()
    lax.fori_loop(0, CH, orow, ())
    iota16 = lax.iota(jnp.int32, L)
    zeros16 = jnp.zeros((L,), jnp.int32)
    def ocol(g):
        plsc.store_scatter(oneslab, [g * L + iota16, zeros16],
                           jnp.full((L,), 1.0, jnp.float32))
    for g in range(CH // L):
        ocol(g)
    plsc.subcore_barrier()

    pltpu.async_copy(dst_p.at[pl.ds(ebase, CH)], idx4.at[0, 0, 1],
                     semi.at[0])

    def deg_step(j, _):
        p = lax.rem(j, 2)
        q = 1 - p
        pltpu.make_async_copy(dst_p.at[pl.ds(0, CH)], idx4.at[p, 0, 1],
                              semi.at[p]).wait()
        @pl.when(j > 0)
        def _():
            pltpu.make_async_copy(oneslab, acc_s.at[idx4.at[q, 0, 1]],
                                  sems.at[q]).wait()
        @pl.when(j < NCH_E - 1)
        def _():
            pltpu.async_copy(dst_p.at[pl.ds(ebase + (j + 1) * CH, CH)],
                             idx4.at[q, 0, 1], semi.at[q])
        pltpu.async_copy(oneslab, acc_s.at[idx4.at[p, 0, 1]],
                         sems.at[p], add=True)
        return ()
    lax.fori_loop(0, NCH_E, deg_step, ())
    pltpu.make_async_copy(oneslab, acc_s.at[idx4.at[(NCH_E - 1) % 2, 0, 1]],
                          sems.at[(NCH_E - 1) % 2]).wait()
    load_tail_idx(dst_p, 1, DUMMY)
    pltpu.sync_copy(oneslab, acc_s.at[idx2.at[1]], add=True)
    plsc.subcore_barrier()

    def dis_chunk(ch):
        pltpu.sync_copy(acc_s.at[pl.ds(ch * CH, CH)], aslab)
        for g in range(CH // L):
            col = plsc.load_gather(aslab, [g * L + iota16, zeros16])
            dbuf[pl.ds(g * L, L)] = _rsqrt16(col)
        pltpu.sync_copy(dbuf, dis_h.at[pl.ds(coff + ch * CH, CH)])
        pltpu.sync_copy(zslab, acc_s.at[pl.ds(ch * CH, CH)])
        lookup_chunk(ch)
    roundrobin(N_CHUNKS, dis_chunk)
    plsc.subcore_barrier()

    def edge_phase():
        def forb(fn):
            def body(b, _):
                fn(b)
                return ()
            lax.fori_loop(0, K, body, ())

        def fire_idx(S, p):
            def f(b):
                base = ebase + (S * K + b) * CH
                pltpu.async_copy(src_p.at[pl.ds(base, CH)],
                                 idx4.at[p, b, 0], semi.at[p])
                pltpu.async_copy(dst_p.at[pl.ds(base, CH)],
                                 idx4.at[p, b, 1], semi.at[p])
            forb(f)

        def drain_idx(p):
            def f(b):
                pltpu.make_async_copy(src_p.at[pl.ds(0, CH)],
                                      idx4.at[p, b, 0], semi.at[p]).wait()
                pltpu.make_async_copy(dst_p.at[pl.ds(0, CH)],
                                      idx4.at[p, b, 1], semi.at[p]).wait()
            forb(f)

        def drain_scat(q):
            def f(b):
                pltpu.make_async_copy(msg4.at[q, b],
                                      acc_s.at[idx4.at[q, b, 1]],
                                      sems.at[q]).wait()
            forb(f)

        fire_idx(0, 0)

        def body(S, _):
            p = lax.rem(S, 2)
            q = 1 - p
            drain_idx(p)
            def off(b):
                for g in range(CH // L):
                    sl = pl.ds(g * L, L)
                    idx4[p, b, 0, sl] = idx4[p, b, 0, sl] + coff
            forb(off)
            forb(lambda b: pltpu.async_copy(y_h.at[idx4.at[p, b, 0]],
                                            msg4.at[p, b], semg.at[p]))
            @pl.when(S > 0)
            def _():
                drain_scat(q)
            @pl.when(S < NSUP - 1)
            def _():
                fire_idx(S + 1, q)
            forb(lambda b: pltpu.make_async_copy(
                y_h.at[idx4.at[p, b, 0]], msg4.at[p, b], semg.at[p]).wait())
            forb(lambda b: pltpu.async_copy(msg4.at[p, b],
                                            acc_s.at[idx4.at[p, b, 1]],
                                            sems.at[p], add=True))
            return ()
        lax.fori_loop(0, NSUP, body, ())
        drain_scat((NSUP - 1) % 2)
        load_tail_idx(src_p, 0, 0)
        load_tail_idx(dst_p, 1, DUMMY)
        for g in range(CH // L):
            sl = pl.ds(g * L, L)
            idx2[0, sl] = idx2[0, sl] + coff
        pltpu.sync_copy(y_h.at[idx2.at[0]], aslab)
        pltpu.sync_copy(aslab, acc_s.at[idx2.at[1]], add=True)

    edge_phase()
    plsc.subcore_barrier()

    fill_zslab()
    def mid_chunk(ch):
        pltpu.sync_copy(acc_s.at[pl.ds(ch * CH, CH)], aslab)
        pltpu.sync_copy(zslab, acc_s.at[pl.ds(ch * CH, CH)])
        load_dis(ch)
        scale_rows(aslab, x0slab, extra=yslab)
        pltpu.sync_copy(x0slab, x1_h.at[pl.ds(coff + ch * CH, CH)])
        pltpu.sync_copy(yslab, y_h.at[pl.ds(coff + ch * CH, CH)])
    roundrobin(N_CHUNKS, mid_chunk)
    plsc.subcore_barrier()

    edge_phase()
    plsc.subcore_barrier()

    def final_chunk(ch):
        pltpu.sync_copy(acc_s.at[pl.ds(ch * CH, CH)], aslab)
        pltpu.sync_copy(x0_h.at[pl.ds(coff + ch * CH, CH)], x0slab)
        pltpu.sync_copy(x1_h.at[pl.ds(coff + ch * CH, CH)], yslab)
        load_dis(ch)
        third = jnp.float32(1.0 / 3.0)
        def row(r, _):
            d = _bcast(dbuf, r)
            for g in range(DH // L):
                sl = pl.ds(g * L, L)
                v = (x0slab[r, sl] + yslab[r, sl] + aslab[r, sl] * d) * third
                zslab[r, sl] = v
            return ()
        lax.fori_loop(0, CH, row, ())
        pltpu.sync_copy(zslab,
                        out_h.at[pl.ds(ch * CH, CH), pl.ds(c * DH, DH)])
    roundrobin(N_CHUNKS, final_chunk)


@jax.jit
def kernel(table, edge_index, node):
    table_r = table.reshape(2 * 1000000, DH)
    src_p = edge_index[0]
    dst_p = edge_index[1]

    mesh = plsc.VectorSubcoreMesh(core_axis_name="c", subcore_axis_name="s")
    run = pl.kernel(
        _body,
        out_type=jax.ShapeDtypeStruct((N_PAD, NC * DH), jnp.float32),
        mesh=mesh,
        compiler_params=pltpu.CompilerParams(needs_layout_passes=False,
                                             use_tc_tiling_on_sc=False),
        scratch_types=[
            pltpu.HBM((NC * N_PAD, DH), jnp.float32),
            pltpu.HBM((NC * N_PAD, DH), jnp.float32),
            pltpu.HBM((NC * N_PAD, DH), jnp.float32),
            pltpu.HBM((NC * N_PAD,), jnp.float32),
            pltpu.VMEM((2, CH), jnp.int32),
            pltpu.VMEM((2, K, 2, CH), jnp.int32),
            pltpu.VMEM((2, K, CH, DH), jnp.float32),
            pltpu.VMEM((CH,), jnp.float32),
            pltpu.VMEM((CH,), jnp.float32),
            pltpu.VMEM((CH,), jnp.float32),
            pltpu.SemaphoreType.DMA((2,)),
            pltpu.SemaphoreType.DMA((2,)),
            pltpu.SemaphoreType.DMA((2,)),
            pltpu.VMEM_SHARED((N_PAD, DH), jnp.float32),
        ],
    )
    o = run(table_r, src_p, dst_p, node)
    return o[:N_NODES_K]
